# Initial kernel scaffold; baseline (speedup 1.0000x reference)
import jax
import jax.numpy as jnp
from jax.experimental import pallas as pl


def kernel(eeg, bio, params):
    raise NotImplementedError("write your pallas kernel here")


# speedup vs baseline: 3.1846x; 3.1846x over previous
import jax
import jax.numpy as jnp
from jax.experimental import pallas as pl
from jax.experimental.pallas import tpu as pltpu

B = 64
V = 40
T = 512
FB = 4
BINS = 10
THRESH = 0.7
SLOPE = 0.01
BN_EPS = 1e-5
NCLS = 3
HID = 64


def _adj_kernel(w_ref, xT_ref, xF_ref, xgT_ref, xgF_ref):
    xt = xT_ref[0]
    mn = jnp.min(xt, axis=1, keepdims=True)
    mx = jnp.max(xt, axis=1, keepdims=True)
    idx = jnp.clip(jnp.floor((xt - mn) / (mx - mn) * BINS).astype(jnp.int32),
                   0, BINS - 1)
    idx_rep = pltpu.repeat(idx, BINS, axis=0)
    p_pat = jax.lax.broadcasted_iota(jnp.int32, (V * BINS, T), 0) // V
    oh = jnp.where(idx_rep == p_pat, 1.0, 0.0).astype(jnp.bfloat16)
    cnt = jax.lax.dot_general(oh, oh, (((1,), (1,)), ((), ())),
                              preferred_element_type=jnp.float32)
    r_i = jax.lax.broadcasted_iota(jnp.int32, (V * BINS, V), 0)
    c_i = jax.lax.broadcasted_iota(jnp.int32, (V * BINS, V), 1)
    S2 = jnp.where(r_i % V == c_i, 1.0, 0.0).astype(jnp.float32)
    ni = jnp.dot(cnt, S2, preferred_element_type=jnp.float32)
    nj = cnt.reshape(BINS, V, V * BINS).sum(axis=0)
    t1f = cnt * jnp.log(jnp.where(cnt > 0, cnt, 1.0))
    t1 = jnp.dot(t1f.reshape(BINS, V, V * BINS).sum(axis=0), S2,
                 preferred_element_type=jnp.float32)
    t2 = (ni * jnp.log(jnp.where(ni > 0, ni, 1.0))).reshape(BINS, V, V).sum(axis=0)
    t3 = jnp.dot(nj * jnp.log(jnp.where(nj > 0, nj, 1.0)), S2,
                 preferred_element_type=jnp.float32)
    N = jnp.float32(T)
    mi = (t1 - t2 - t3) / N + jnp.log(N)
    mi = jnp.maximum(mi, 0.0)
    mi = jnp.where(jnp.isfinite(mi), mi, 0.0)
    adj = jnp.where(mi < THRESH, 0.0, mi)
    eye = (jax.lax.broadcasted_iota(jnp.int32, (V, V), 0)
           == jax.lax.broadcasted_iota(jnp.int32, (V, V), 1))
    H = jnp.where(eye, 1.0, adj)
    dinv_c = jax.lax.rsqrt(jnp.sum(H, axis=1, keepdims=True))
    dinv_r = jax.lax.rsqrt(jnp.sum(H, axis=0, keepdims=True))
    Hn = H * dinv_c * dinv_r
    wF = w_ref[0]
    wT = w_ref[1]
    xgT_ref[0] = jax.nn.leaky_relu(
        jnp.dot(Hn, xt, preferred_element_type=jnp.float32) * wT, SLOPE)
    xf = xF_ref[0]
    xgF_ref[0] = jax.nn.leaky_relu(
        jnp.dot(Hn, xf, preferred_element_type=jnp.float32) * wF, SLOPE)


def _make_gru(L):
    def _gru_kernel(xs_ref, wih_ref, whh_ref, bih_ref, bhh_ref, y_ref):
        d = pl.program_id(0)
        wih = wih_ref[0]
        whh = whh_ref[0]
        bih = bih_ref[0]
        bhh = bhh_ref[0]

        def step(s, h):
            t = jax.lax.select(d == 0, s, L - 1 - s)
            xt = xs_ref[t]
            gi = jnp.dot(xt, wih, preferred_element_type=jnp.float32) + bih
            gh = jnp.dot(h, whh, preferred_element_type=jnp.float32) + bhh
            r = jax.nn.sigmoid(gi[:, 0:V] + gh[:, 0:V])
            z = jax.nn.sigmoid(gi[:, V:2 * V] + gh[:, V:2 * V])
            n = jnp.tanh(gi[:, 2 * V:3 * V] + r * gh[:, 2 * V:3 * V])
            hn = (1.0 - z) * n + z * h
            y_ref[0, pl.ds(t, 1)] = hn[None]
            return hn

        jax.lax.fori_loop(0, L, step, jnp.zeros((B, V), jnp.float32))

    return _gru_kernel


def _head_kernel(zT_ref, zF_ref, gT_ref, bT_ref, gF_ref, bF_ref,
                 wTt_ref, bTl_ref, wFt_ref, bFl_ref,
                 w1T_ref, w1F_ref, b1_ref, w2t_ref, b2_ref, out_ref):
    def bn_proj(z, g, b, wt, bl):
        mu = jnp.mean(z, axis=0, keepdims=True)
        var = jnp.mean(z * z, axis=0, keepdims=True) - mu * mu
        yn = (z - mu) * jax.lax.rsqrt(var + BN_EPS) * g + b
        return jnp.dot(yn, wt, preferred_element_type=jnp.float32) + bl

    oT = bn_proj(zT_ref[...], gT_ref[...], bT_ref[...], wTt_ref[...],
                 bTl_ref[...])
    oF = bn_proj(zF_ref[...], gF_ref[...], bF_ref[...], wFt_ref[...],
                 bFl_ref[...])
    acc = jnp.zeros((B, HID), jnp.float32)
    for dh in range(2 * V):
        blkF = oF[dh * B:(dh + 1) * B, :]
        blkT = oT[dh * B:(dh + 1) * B, :]
        acc = acc + jnp.dot(blkF, w1F_ref[dh], preferred_element_type=jnp.float32)
        acc = acc + jnp.dot(blkT, w1T_ref[dh], preferred_element_type=jnp.float32)
    out1 = jax.nn.leaky_relu(acc + b1_ref[...], SLOPE)
    out_ref[...] = jnp.dot(out1, w2t_ref[...],
                           preferred_element_type=jnp.float32) + b2_ref[...]


def _gru_call(xs, p, L):
    wih = jnp.stack([p['Wih_f'].T, p['Wih_b'].T])
    whh = jnp.stack([p['Whh_f'].T, p['Whh_b'].T])
    bih = jnp.stack([p['bih_f'][None], p['bih_b'][None]])
    bhh = jnp.stack([p['bhh_f'][None], p['bhh_b'][None]])
    return pl.pallas_call(
        _make_gru(L),
        grid=(2,),
        in_specs=[
            pl.BlockSpec((L, B, V), lambda d: (0, 0, 0)),
            pl.BlockSpec((1, V, 3 * V), lambda d: (d, 0, 0)),
            pl.BlockSpec((1, V, 3 * V), lambda d: (d, 0, 0)),
            pl.BlockSpec((1, 1, 3 * V), lambda d: (d, 0, 0)),
            pl.BlockSpec((1, 1, 3 * V), lambda d: (d, 0, 0)),
        ],
        out_specs=pl.BlockSpec((1, L, B, V), lambda d: (d, 0, 0, 0)),
        out_shape=jax.ShapeDtypeStruct((2, L, B, V), jnp.float32),
        compiler_params=pltpu.CompilerParams(
            dimension_semantics=("parallel",),
            vmem_limit_bytes=56 * 1024 * 1024,
        ),
    )(xs, wih, whh, bih, bhh)


def kernel(eeg, bio, params):
    x = jnp.concatenate([eeg, bio], 1)
    x_F = x[:, :, :FB]
    x_T = x[:, :, FB:]
    wvec = jnp.stack([params['F']['w'][0, 0], params['T']['w'][0, 0]])

    xgT, xgF = pl.pallas_call(
        _adj_kernel,
        grid=(B,),
        in_specs=[
            pl.BlockSpec(memory_space=pltpu.SMEM),
            pl.BlockSpec((1, V, T), lambda b: (b, 0, 0)),
            pl.BlockSpec((1, V, FB), lambda b: (b, 0, 0)),
        ],
        out_specs=[
            pl.BlockSpec((1, V, T), lambda b: (b, 0, 0)),
            pl.BlockSpec((1, V, FB), lambda b: (b, 0, 0)),
        ],
        out_shape=[
            jax.ShapeDtypeStruct((B, V, T), jnp.float32),
            jax.ShapeDtypeStruct((B, V, FB), jnp.float32),
        ],
        compiler_params=pltpu.CompilerParams(
            dimension_semantics=("parallel",),
            vmem_limit_bytes=56 * 1024 * 1024,
        ),
    )(wvec, x_T, x_F)

    YT = _gru_call(xgT.transpose(2, 0, 1), params['T'], T)
    YF = _gru_call(xgF.transpose(2, 0, 1), params['F'], FB)

    zT = YT.transpose(0, 3, 2, 1).reshape(2 * V * B, T)
    zF = YF.transpose(0, 3, 2, 1).reshape(2 * V * B, FB)
    w1 = params['W1'].reshape(HID, 2, 2 * V, T // 2)
    w1F = w1[:, 0].transpose(1, 2, 0)
    w1T = w1[:, 1].transpose(1, 2, 0)

    out = pl.pallas_call(
        _head_kernel,
        out_shape=jax.ShapeDtypeStruct((B, NCLS), jnp.float32),
        compiler_params=pltpu.CompilerParams(
            vmem_limit_bytes=56 * 1024 * 1024,
        ),
    )(zT, zF,
      params['T']['gamma'][None], params['T']['beta'][None],
      params['F']['gamma'][None], params['F']['beta'][None],
      params['WT'].T, params['bT'][None], params['WF'].T, params['bF'][None],
      w1T, w1F, params['b1'][None], params['W2'].T, params['b2'][None])
    return out


# speedup vs baseline: 5.1882x; 1.6292x over previous
import jax
import jax.numpy as jnp
from jax.experimental import pallas as pl
from jax.experimental.pallas import tpu as pltpu

B = 64
V = 40
T = 512
FB = 4
BINS = 10
THRESH = 0.7
SLOPE = 0.01
BN_EPS = 1e-5
NCLS = 3
HID = 64


def _adj_kernel(w_ref, xT_ref, xF_ref, xgT_ref, xgF_ref):
    xt = xT_ref[0]
    xf = xF_ref[0]
    mn = jnp.min(xt, axis=1, keepdims=True)
    mx = jnp.max(xt, axis=1, keepdims=True)
    idx = jnp.clip(jnp.floor((xt - mn) / (mx - mn) * BINS).astype(jnp.int32),
                   0, BINS - 1)
    idx_rep = pltpu.repeat(idx, BINS, axis=0)
    p_pat = jax.lax.broadcasted_iota(jnp.int32, (V * BINS, T), 0) // V
    oh = jnp.where(idx_rep == p_pat, 1.0, 0.0).astype(jnp.bfloat16)
    cnt = jax.lax.dot_general(oh, oh, (((1,), (1,)), ((), ())),
                              preferred_element_type=jnp.float32)
    r_i = jax.lax.broadcasted_iota(jnp.int32, (V * BINS, V), 0)
    c_i = jax.lax.broadcasted_iota(jnp.int32, (V * BINS, V), 1)
    S2 = jnp.where(r_i % V == c_i, 1.0, 0.0).astype(jnp.float32)
    ni = jnp.dot(cnt, S2, preferred_element_type=jnp.float32)
    nj = cnt.reshape(BINS, V, V * BINS).sum(axis=0)
    t1f = cnt * jnp.log(jnp.where(cnt > 0, cnt, 1.0))
    t1 = jnp.dot(t1f.reshape(BINS, V, V * BINS).sum(axis=0), S2,
                 preferred_element_type=jnp.float32)
    t2 = (ni * jnp.log(jnp.where(ni > 0, ni, 1.0))).reshape(BINS, V, V).sum(axis=0)
    t3 = jnp.dot(nj * jnp.log(jnp.where(nj > 0, nj, 1.0)), S2,
                 preferred_element_type=jnp.float32)
    N = jnp.float32(T)
    mi = (t1 - t2 - t3) / N + jnp.log(N)
    mi = jnp.maximum(mi, 0.0)
    mi = jnp.where(jnp.isfinite(mi), mi, 0.0)
    adj = jnp.where(mi < THRESH, 0.0, mi)
    eye = (jax.lax.broadcasted_iota(jnp.int32, (V, V), 0)
           == jax.lax.broadcasted_iota(jnp.int32, (V, V), 1))
    H = jnp.where(eye, 1.0, adj)
    dinv_c = jax.lax.rsqrt(jnp.sum(H, axis=1, keepdims=True))
    dinv_r = jax.lax.rsqrt(jnp.sum(H, axis=0, keepdims=True))
    Hn = H * dinv_c * dinv_r
    wF = w_ref[0]
    wT = w_ref[1]
    xgT_ref[0] = jax.nn.leaky_relu(
        jnp.dot(Hn, xt, preferred_element_type=jnp.float32) * wT, SLOPE)
    xgF_ref[0] = jax.nn.leaky_relu(
        jnp.dot(Hn, xf, preferred_element_type=jnp.float32) * wF, SLOPE)


def _make_gru(L):
    Bh = B // 2

    def _gru_kernel(xs_ref, wihP_ref, whhP_ref, bi_ref, bhn_ref, y_ref):
        wf, wb = wihP_ref[0], wihP_ref[1]
        whf, whb = whhP_ref[0], whhP_ref[1]
        bif, bib = bi_ref[0], bi_ref[1]
        bhnf, bhnb = bhn_ref[0], bhn_ref[1]

        Q = 8
        NC = Bh // Q

        def gates(gi, h, wh, bhn):
            gh = jnp.dot(h, wh, preferred_element_type=jnp.float32)
            r = jax.nn.sigmoid(gi[:, 0:V] + gh[:, 0:V])
            z = jax.nn.sigmoid(gi[:, 128:128 + V] + gh[:, 128:128 + V])
            n = jnp.tanh(gi[:, 256:256 + V] + r * (gh[:, 256:256 + V] + bhn))
            return (1.0 - z) * n + z * h

        def step(s, carry):
            hfs, hbs = carry
            tb = L - 1 - s
            gif = jnp.dot(xs_ref[s], wf,
                          preferred_element_type=jnp.float32) + bif
            gib = jnp.dot(xs_ref[tb], wb,
                          preferred_element_type=jnp.float32) + bib
            hfs = tuple(gates(gif[k * Q:(k + 1) * Q], hfs[k], whf, bhnf)
                        for k in range(NC))
            hbs = tuple(gates(gib[k * Q:(k + 1) * Q], hbs[k], whb, bhnb)
                        for k in range(NC))
            y_ref[0, pl.ds(s, 1)] = jnp.concatenate(hfs, 0)[None]
            y_ref[1, pl.ds(tb, 1)] = jnp.concatenate(hbs, 0)[None]
            return (hfs, hbs)

        h0 = jnp.zeros((Q, V), jnp.float32)
        jax.lax.fori_loop(0, L, step, ((h0,) * NC, (h0,) * NC))

    return _gru_kernel


def _head_kernel(zT_ref, zF_ref, gT_ref, bT_ref, gF_ref, bF_ref,
                 wTt_ref, bTl_ref, wFt_ref, bFl_ref,
                 w1_ref, b1_ref, w2t_ref, b2_ref, out_ref):
    def bn_proj(z, g, b, wt, bl):
        mu = jnp.mean(z, axis=0, keepdims=True)
        var = jnp.mean(z * z, axis=0, keepdims=True) - mu * mu
        yn = (z - mu) * jax.lax.rsqrt(var + BN_EPS) * g + b
        return jnp.dot(yn, wt, preferred_element_type=jnp.float32) + bl

    oT = bn_proj(zT_ref[...], gT_ref[...], bT_ref[...], wTt_ref[...],
                 bTl_ref[...])
    oF = bn_proj(zF_ref[...], gF_ref[...], bF_ref[...], wFt_ref[...],
                 bFl_ref[...])
    O = T // 2
    half = 2 * V * O
    acc = jnp.zeros((B, HID), jnp.float32)
    dn = (((1,), (1,)), ((), ()))
    for dh in range(2 * V):
        blkF = oF[dh * B:(dh + 1) * B, :]
        blkT = oT[dh * B:(dh + 1) * B, :]
        acc = acc + jax.lax.dot_general(
            blkF, w1_ref[:, dh * O:(dh + 1) * O], dn,
            preferred_element_type=jnp.float32)
        acc = acc + jax.lax.dot_general(
            blkT, w1_ref[:, half + dh * O:half + (dh + 1) * O], dn,
            preferred_element_type=jnp.float32)
    out1 = jax.nn.leaky_relu(acc + b1_ref[...], SLOPE)
    out_ref[...] = jnp.dot(out1, w2t_ref[...],
                           preferred_element_type=jnp.float32) + b2_ref[...]


def _gru_call(xs, p, L):
    def pad_gates(w):
        wP = jnp.zeros((V, 384), jnp.float32)
        wP = wP.at[:, 0:V].set(w[:, 0:V])
        wP = wP.at[:, 128:128 + V].set(w[:, V:2 * V])
        wP = wP.at[:, 256:256 + V].set(w[:, 2 * V:3 * V])
        return wP

    def per_dir(wih, whh, bih, bhh):
        bi = jnp.zeros((1, 384), jnp.float32)
        bi = bi.at[0, 0:V].set(bih[0:V] + bhh[0:V])
        bi = bi.at[0, 128:128 + V].set(bih[V:2 * V] + bhh[V:2 * V])
        bi = bi.at[0, 256:256 + V].set(bih[2 * V:3 * V])
        return (pad_gates(wih.T), pad_gates(whh.T), bi,
                bhh[2 * V:3 * V][None])

    f = per_dir(p['Wih_f'], p['Whh_f'], p['bih_f'], p['bhh_f'])
    b = per_dir(p['Wih_b'], p['Whh_b'], p['bih_b'], p['bhh_b'])
    stacked = [jnp.stack([pf, pb]) for pf, pb in zip(f, b)]

    def dspec(shape):
        return pl.BlockSpec((2,) + shape, lambda d: (0, 0, 0))

    return pl.pallas_call(
        _make_gru(L),
        grid=(2,),
        in_specs=[
            pl.BlockSpec((L, B // 2, V), lambda d: (0, d, 0)),
            dspec((V, 384)), dspec((V, 384)),
            dspec((1, 384)), dspec((1, V)),
        ],
        out_specs=pl.BlockSpec((2, L, B // 2, V), lambda d: (0, 0, d, 0)),
        out_shape=jax.ShapeDtypeStruct((2, L, B, V), jnp.float32),
        compiler_params=pltpu.CompilerParams(
            dimension_semantics=("parallel",),
            vmem_limit_bytes=56 * 1024 * 1024,
        ),
    )(xs, *stacked)


def kernel(eeg, bio, params):
    x = jnp.concatenate([eeg, bio], 1)
    x_F = x[:, :, :FB]
    x_T = x[:, :, FB:]
    wvec = jnp.stack([params['F']['w'][0, 0], params['T']['w'][0, 0]])

    xgT, xgF = pl.pallas_call(
        _adj_kernel,
        grid=(B,),
        in_specs=[
            pl.BlockSpec(memory_space=pltpu.SMEM),
            pl.BlockSpec((1, V, T), lambda b: (b, 0, 0)),
            pl.BlockSpec((1, V, FB), lambda b: (b, 0, 0)),
        ],
        out_specs=[
            pl.BlockSpec((1, V, T), lambda b: (b, 0, 0)),
            pl.BlockSpec((1, V, FB), lambda b: (b, 0, 0)),
        ],
        out_shape=[
            jax.ShapeDtypeStruct((B, V, T), jnp.float32),
            jax.ShapeDtypeStruct((B, V, FB), jnp.float32),
        ],
        compiler_params=pltpu.CompilerParams(
            dimension_semantics=("parallel",),
            vmem_limit_bytes=56 * 1024 * 1024,
        ),
    )(wvec, x_T, x_F)

    YT = _gru_call(xgT.transpose(2, 0, 1), params['T'], T)
    YF = _gru_call(xgF.transpose(2, 0, 1), params['F'], FB)

    zT = YT.transpose(0, 3, 2, 1).reshape(2 * V * B, T)
    zF = YF.transpose(0, 3, 2, 1).reshape(2 * V * B, FB)

    out = pl.pallas_call(
        _head_kernel,
        out_shape=jax.ShapeDtypeStruct((B, NCLS), jnp.float32),
        compiler_params=pltpu.CompilerParams(
            vmem_limit_bytes=56 * 1024 * 1024,
        ),
    )(zT, zF,
      params['T']['gamma'][None], params['T']['beta'][None],
      params['F']['gamma'][None], params['F']['beta'][None],
      params['WT'].T, params['bT'][None], params['WF'].T, params['bF'][None],
      params['W1'], params['b1'][None], params['W2'].T, params['b2'][None])
    return out
